# Initial kernel scaffold; baseline (speedup 1.0000x reference)
import jax
import jax.numpy as jnp
from jax.experimental import pallas as pl


def kernel(coords, hash_tables, W0, b0, W1, b1, W2, b2, W3, b3):
    raise NotImplementedError("write your pallas kernel here")


# speedup vs baseline: 1.9506x; 1.9506x over previous
import functools

import numpy as np
import jax
import jax.numpy as jnp
from jax import lax
from jax.experimental import pallas as pl
from jax.experimental.pallas import tpu as pltpu
from jax.experimental.pallas import tpu_sc as plsc

N_POINTS = 524288
N_LEVELS = 16
F_PER = 2
HASHMAP = 2 ** 19
MASK = np.int32(HASHMAP - 1)
BASE = 16
FINEST = 512
_b = np.exp((np.log(FINEST) - np.log(BASE)) / (N_LEVELS - 1))
RESOLUTIONS = np.array([int(np.ceil(BASE * _b ** i)) for i in range(N_LEVELS)],
                       dtype=np.float32)
P1 = np.int32(np.uint32(2654435761 & 0xFFFFFFFF))
P2 = np.int32(805459861)
CLIP_HI = np.float32(1.0 - 1e-06)

NC = 2
NS = 16
NW = NC * NS
PW = N_POINTS // NW
C = 128
NCHUNK = PW // C
G16 = C // 16

F32 = jnp.float32
I32 = jnp.int32


def _i32(x):
    return jnp.int32(x)


def _enc_body(resrep_hbm, xs_hbm, ys_hbm, zs_hbm, table_hbm, enc_hbm,
              resrep, xv, yv, zv, idxb, wb, rows, encb, sem):
    cid = lax.axis_index("c").astype(I32)
    sid = lax.axis_index("s").astype(I32)
    wid = sid * _i32(NC) + cid
    pltpu.sync_copy(resrep_hbm, resrep)
    base0 = wid * _i32(PW)

    iota16 = jnp.arange(16, dtype=I32)
    zeros16 = jnp.zeros((16,), I32)
    ones16 = jnp.ones((16,), I32)

    def chunk_body(ci, carry):
        base = base0 + ci * _i32(C)
        pltpu.sync_copy(xs_hbm.at[pl.ds(base, C)], xv)
        pltpu.sync_copy(ys_hbm.at[pl.ds(base, C)], yv)
        pltpu.sync_copy(zs_hbm.at[pl.ds(base, C)], zv)

        def level_body(l, carry2):
            res_b = resrep[l, :]
            loff = l * _i32(HASHMAP)

            def hash_group(pg, carry3):
                sl = pl.ds(pg * _i32(16), 16)
                x = jnp.clip(xv[sl], F32(0.0), CLIP_HI)
                y = jnp.clip(yv[sl], F32(0.0), CLIP_HI)
                z = jnp.clip(zv[sl], F32(0.0), CLIP_HI)
                sx = x * res_b
                sy = y * res_b
                sz = z * res_b
                ix = sx.astype(I32)
                iy = sy.astype(I32)
                iz = sz.astype(I32)
                wx1 = sx - ix.astype(F32)
                wy1 = sy - iy.astype(F32)
                wz1 = sz - iz.astype(F32)
                wx0 = F32(1.0) - wx1
                wy0 = F32(1.0) - wy1
                wz0 = F32(1.0) - wz1
                hx = (ix, ix + _i32(1))
                hy = (iy * P1, iy * P1 + P1)
                hz = (iz * P2, iz * P2 + P2)
                wx = (wx0, wx1)
                wy = (wy0, wy1)
                wz = (wz0, wz1)
                for i in range(2):
                    for j in range(2):
                        for k in range(2):
                            corner = np.int32(i * 4 + j * 2 + k)
                            hidx = ((hx[i] ^ hy[j] ^ hz[k]) & MASK) + loff
                            idxb[corner, sl] = hidx
                            wb[corner, sl] = wx[i] * wy[j] * wz[k]
                return carry3

            lax.fori_loop(_i32(0), _i32(G16), hash_group, _i32(0))

            copies = [pltpu.async_copy(table_hbm.at[idxb.at[np.int32(corner)]],
                                       rows.at[np.int32(corner)], sem)
                      for corner in range(8)]
            for cp in copies:
                cp.wait()

            def interp_group(pg, carry3):
                sl = pl.ds(pg * _i32(16), 16)
                p_idx = pg * _i32(16) + iota16
                acc0 = jnp.zeros((16,), F32)
                acc1 = jnp.zeros((16,), F32)
                for corner in range(8):
                    csplat = jnp.full((16,), corner, I32)
                    f0 = plsc.load_gather(rows, [csplat, p_idx, zeros16])
                    f1 = plsc.load_gather(rows, [csplat, p_idx, ones16])
                    wwv = wb[np.int32(corner), sl]
                    acc0 = acc0 + wwv * f0
                    acc1 = acc1 + wwv * f1
                encb[l * _i32(2), sl] = acc0
                encb[l * _i32(2) + _i32(1), sl] = acc1
                return carry3

            lax.fori_loop(_i32(0), _i32(G16), interp_group, _i32(0))
            return carry2

        lax.fori_loop(_i32(0), _i32(N_LEVELS), level_body, _i32(0))
        pltpu.sync_copy(encb, enc_hbm.at[:, pl.ds(base, C)])
        return carry

    lax.fori_loop(_i32(0), _i32(NCHUNK), chunk_body, _i32(0))


_enc_call = functools.partial(
    pl.kernel,
    out_type=jax.ShapeDtypeStruct((2 * N_LEVELS, N_POINTS), jnp.float32),
    mesh=plsc.VectorSubcoreMesh(core_axis_name="c", subcore_axis_name="s"),
    compiler_params=pltpu.CompilerParams(needs_layout_passes=False,
                                         use_tc_tiling_on_sc=False),
    scratch_types=[
        pltpu.VMEM((N_LEVELS, 16), F32),
        pltpu.VMEM((C,), F32),
        pltpu.VMEM((C,), F32),
        pltpu.VMEM((C,), F32),
        pltpu.VMEM((8, C), I32),
        pltpu.VMEM((8, C), F32),
        pltpu.VMEM((8, C, F_PER), F32),
        pltpu.VMEM((2 * N_LEVELS, C), F32),
        pltpu.SemaphoreType.DMA,
    ],
)(_enc_body)


B_MLP = 2048


def _mlp_body(enc_ref, w0, b0, w1, b1, w2, b2, w3, b3, out_ref):
    h = jnp.dot(w0[...], enc_ref[...], preferred_element_type=F32) + b0[...]
    h = jnp.maximum(h, F32(0.0))
    h = jnp.dot(w1[...], h, preferred_element_type=F32) + b1[...]
    h = jnp.maximum(h, F32(0.0))
    h = jnp.dot(w2[...], h, preferred_element_type=F32) + b2[...]
    h = jnp.maximum(h, F32(0.0))
    o = jnp.dot(w3[...], h, preferred_element_type=F32) + b3[...]
    out_ref[...] = jax.nn.sigmoid(o)


IN_DIM = 2 * N_LEVELS
HIDDEN = 64

_Z = np.int32(0)


def _col_map(i):
    return (_Z, i)


def _fix_map(i):
    return (_Z, _Z)


_mlp_call = pl.pallas_call(
    _mlp_body,
    grid=(N_POINTS // B_MLP,),
    in_specs=[
        pl.BlockSpec((IN_DIM, B_MLP), _col_map),
        pl.BlockSpec((HIDDEN, IN_DIM), _fix_map),
        pl.BlockSpec((HIDDEN, 1), _fix_map),
        pl.BlockSpec((HIDDEN, HIDDEN), _fix_map),
        pl.BlockSpec((HIDDEN, 1), _fix_map),
        pl.BlockSpec((HIDDEN, HIDDEN), _fix_map),
        pl.BlockSpec((HIDDEN, 1), _fix_map),
        pl.BlockSpec((1, HIDDEN), _fix_map),
        pl.BlockSpec((1, 1), _fix_map),
    ],
    out_specs=pl.BlockSpec((1, B_MLP), _col_map),
    out_shape=jax.ShapeDtypeStruct((1, N_POINTS), jnp.float32),
)


def kernel(coords, hash_tables, W0, b0, W1, b1, W2, b2, W3, b3):
    coords = coords.astype(jnp.float32)
    xs = coords[:, 0]
    ys = coords[:, 1]
    zs = coords[:, 2]
    table2 = hash_tables.astype(jnp.float32).reshape(N_LEVELS * HASHMAP, F_PER)
    resrep = jnp.broadcast_to(
        jnp.asarray(RESOLUTIONS)[:, None], (N_LEVELS, 16)).astype(jnp.float32)
    enc = _enc_call(resrep, xs, ys, zs, table2)
    out = _mlp_call(enc,
                    W0.T.astype(jnp.float32), b0[:, None].astype(jnp.float32),
                    W1.T.astype(jnp.float32), b1[:, None].astype(jnp.float32),
                    W2.T.astype(jnp.float32), b2[:, None].astype(jnp.float32),
                    W3.T.astype(jnp.float32), b3[:, None].astype(jnp.float32))
    return out.reshape(N_POINTS, 1)


# speedup vs baseline: 6.7500x; 3.4606x over previous
import functools

import numpy as np
import jax
import jax.numpy as jnp
from jax import lax
from jax.experimental import pallas as pl
from jax.experimental.pallas import tpu as pltpu
from jax.experimental.pallas import tpu_sc as plsc

N_POINTS = 524288
N_LEVELS = 16
F_PER = 2
HASHMAP = 2 ** 19
MASK = np.int32(HASHMAP - 1)
BASE = 16
FINEST = 512
_b = np.exp((np.log(FINEST) - np.log(BASE)) / (N_LEVELS - 1))
RESOLUTIONS = np.array([int(np.ceil(BASE * _b ** i)) for i in range(N_LEVELS)],
                       dtype=np.float32)
P1 = np.int32(np.uint32(2654435761 & 0xFFFFFFFF))
P2 = np.int32(805459861)
CLIP_HI = np.float32(1.0 - 1e-06)

NC = 2
NS = 16
NW = NC * NS
PW = N_POINTS // NW
C = 128
NCHUNK = PW // C
G16 = C // 16

TBLW = N_LEVELS * HASHMAP * F_PER
BPW = TBLW // NW
IBLK = 4096
NIB = BPW // IBLK

F32 = jnp.float32
I32 = jnp.int32


def _i32(x):
    return jnp.int32(x)


def _worker_id():
    cid = lax.axis_index("c").astype(I32)
    sid = lax.axis_index("s").astype(I32)
    return sid * _i32(NC) + cid


def _intl_body(tn_hbm, tout_hbm, buf, obuf):
    wid = _worker_id()
    woff0 = wid * _i32(BPW)
    iota16 = jnp.arange(16, dtype=I32)
    zeros16 = jnp.zeros((16,), I32)
    ones16 = jnp.ones((16,), I32)

    def ib(i, carry):
        woff = woff0 + i * _i32(IBLK)
        pltpu.sync_copy(tn_hbm.at[pl.ds(woff, IBLK)], buf)

        def grp(q, carry2):
            s0 = lax.shift_right_logical(q, _i32(3)) * _i32(256) \
                + (q & _i32(7)) * _i32(16)
            f0 = buf[pl.ds(s0, 16)]
            f1 = buf[pl.ds(s0 + _i32(128), 16)]
            eidx = q * _i32(16) + iota16
            plsc.store_scatter(obuf, [eidx, zeros16], f0)
            plsc.store_scatter(obuf, [eidx, ones16], f1)
            return carry2

        lax.fori_loop(_i32(0), _i32(IBLK // 32), grp, _i32(0))
        ebase = lax.shift_right_logical(woff, _i32(1))
        pltpu.sync_copy(obuf, tout_hbm.at[pl.ds(ebase, IBLK // 2)])
        return carry

    lax.fori_loop(_i32(0), _i32(NIB), ib, _i32(0))


_intl_call = functools.partial(
    pl.kernel,
    out_type=jax.ShapeDtypeStruct((N_LEVELS * HASHMAP, F_PER), jnp.float32),
    mesh=plsc.VectorSubcoreMesh(core_axis_name="c", subcore_axis_name="s"),
    compiler_params=pltpu.CompilerParams(needs_layout_passes=False,
                                         use_tc_tiling_on_sc=False),
    scratch_types=[
        pltpu.VMEM((IBLK,), F32),
        pltpu.VMEM((IBLK // 2, F_PER), F32),
    ],
)(_intl_body)


def _enc_body(resrep_hbm, xs_hbm, ys_hbm, zs_hbm, table_hbm, enc_hbm,
              resrep, xv, yv, zv, idxb, wb, rows, encb, sem, osem):
    wid = _worker_id()
    pltpu.sync_copy(resrep_hbm, resrep)
    base0 = wid * _i32(PW)
    ct0 = wid * _i32(NCHUNK)

    iota16 = jnp.arange(16, dtype=I32)
    zeros16 = jnp.zeros((16,), I32)
    ones16 = jnp.ones((16,), I32)

    def chunk_body(ci, carry):
        base = base0 + ci * _i32(C)
        pltpu.sync_copy(xs_hbm.at[pl.ds(base, C)], xv)
        pltpu.sync_copy(ys_hbm.at[pl.ds(base, C)], yv)
        pltpu.sync_copy(zs_hbm.at[pl.ds(base, C)], zv)

        def level_body(l, carry2):
            res_b = resrep[l, :]
            loff = l * _i32(HASHMAP)

            def hash_group(pg, carry3):
                sl = pl.ds(pg * _i32(16), 16)
                x = jnp.clip(xv[sl], F32(0.0), CLIP_HI)
                y = jnp.clip(yv[sl], F32(0.0), CLIP_HI)
                z = jnp.clip(zv[sl], F32(0.0), CLIP_HI)
                sx = x * res_b
                sy = y * res_b
                sz = z * res_b
                ix = sx.astype(I32)
                iy = sy.astype(I32)
                iz = sz.astype(I32)
                wx1 = sx - ix.astype(F32)
                wy1 = sy - iy.astype(F32)
                wz1 = sz - iz.astype(F32)
                wx0 = F32(1.0) - wx1
                wy0 = F32(1.0) - wy1
                wz0 = F32(1.0) - wz1
                hx = (ix, ix + _i32(1))
                hy = (iy * P1, iy * P1 + P1)
                hz = (iz * P2, iz * P2 + P2)
                wx = (wx0, wx1)
                wy = (wy0, wy1)
                wz = (wz0, wz1)
                for i in range(2):
                    for j in range(2):
                        for k in range(2):
                            corner = np.int32(i * 4 + j * 2 + k)
                            hidx = ((hx[i] ^ hy[j] ^ hz[k]) & MASK) + loff
                            idxb[corner, sl] = hidx
                            wb[corner, sl] = wx[i] * wy[j] * wz[k]
                return carry3

            lax.fori_loop(_i32(0), _i32(G16), hash_group, _i32(0))

            copies = [pltpu.async_copy(table_hbm.at[idxb.at[np.int32(corner)]],
                                       rows.at[np.int32(corner)], sem)
                      for corner in range(8)]
            for cp in copies:
                cp.wait()

            def interp_group(pg, carry3):
                sl = pl.ds(pg * _i32(16), 16)
                p_idx = pg * _i32(16) + iota16
                acc0 = jnp.zeros((16,), F32)
                acc1 = jnp.zeros((16,), F32)
                for corner in range(8):
                    csplat = jnp.full((16,), corner, I32)
                    f0 = plsc.load_gather(rows, [csplat, p_idx, zeros16])
                    f1 = plsc.load_gather(rows, [csplat, p_idx, ones16])
                    wwv = wb[np.int32(corner), sl]
                    acc0 = acc0 + wwv * f0
                    acc1 = acc1 + wwv * f1
                encb[l * _i32(2), sl] = acc0
                encb[l * _i32(2) + _i32(1), sl] = acc1
                return carry3

            lax.fori_loop(_i32(0), _i32(G16), interp_group, _i32(0))
            return carry2

        lax.fori_loop(_i32(0), _i32(N_LEVELS), level_body, _i32(0))
        ct = ct0 + ci
        ocopies = [pltpu.async_copy(encb.at[pl.ds(np.int32(8 * t), 8)],
                                    enc_hbm.at[np.int32(t), ct], osem)
                   for t in range(4)]
        for cp in ocopies:
            cp.wait()
        return carry

    lax.fori_loop(_i32(0), _i32(NCHUNK), chunk_body, _i32(0))


_enc_call = functools.partial(
    pl.kernel,
    out_type=jax.ShapeDtypeStruct((4, N_POINTS // 128, 8, 128), jnp.float32),
    mesh=plsc.VectorSubcoreMesh(core_axis_name="c", subcore_axis_name="s"),
    compiler_params=pltpu.CompilerParams(needs_layout_passes=False,
                                         use_tc_tiling_on_sc=False),
    scratch_types=[
        pltpu.VMEM((N_LEVELS, 16), F32),
        pltpu.VMEM((C,), F32),
        pltpu.VMEM((C,), F32),
        pltpu.VMEM((C,), F32),
        pltpu.VMEM((8, C), I32),
        pltpu.VMEM((8, C), F32),
        pltpu.VMEM((8, C, F_PER), F32),
        pltpu.VMEM((2 * N_LEVELS, C), F32),
        pltpu.SemaphoreType.DMA,
        pltpu.SemaphoreType.DMA,
    ],
)(_enc_body)


B_MLP = 2048


def _mlp_body(enc_ref, w0, b0, w1, b1, w2, b2, w3, b3, out_ref):
    h = jnp.dot(w0[...], enc_ref[...], preferred_element_type=F32) + b0[...]
    h = jnp.maximum(h, F32(0.0))
    h = jnp.dot(w1[...], h, preferred_element_type=F32) + b1[...]
    h = jnp.maximum(h, F32(0.0))
    h = jnp.dot(w2[...], h, preferred_element_type=F32) + b2[...]
    h = jnp.maximum(h, F32(0.0))
    o = jnp.dot(w3[...], h, preferred_element_type=F32) + b3[...]
    out_ref[...] = jax.nn.sigmoid(o)


IN_DIM = 2 * N_LEVELS
HIDDEN = 64

_Z = np.int32(0)


def _col_map(i):
    return (_Z, i)


def _fix_map(i):
    return (_Z, _Z)


_mlp_call = pl.pallas_call(
    _mlp_body,
    grid=(N_POINTS // B_MLP,),
    in_specs=[
        pl.BlockSpec((IN_DIM, B_MLP), _col_map),
        pl.BlockSpec((HIDDEN, IN_DIM), _fix_map),
        pl.BlockSpec((HIDDEN, 1), _fix_map),
        pl.BlockSpec((HIDDEN, HIDDEN), _fix_map),
        pl.BlockSpec((HIDDEN, 1), _fix_map),
        pl.BlockSpec((HIDDEN, HIDDEN), _fix_map),
        pl.BlockSpec((HIDDEN, 1), _fix_map),
        pl.BlockSpec((1, HIDDEN), _fix_map),
        pl.BlockSpec((1, 1), _fix_map),
    ],
    out_specs=pl.BlockSpec((1, B_MLP), _col_map),
    out_shape=jax.ShapeDtypeStruct((1, N_POINTS), jnp.float32),
)


def kernel(coords, hash_tables, W0, b0, W1, b1, W2, b2, W3, b3):
    coords = coords.astype(jnp.float32)
    xs = coords[:, 0]
    ys = coords[:, 1]
    zs = coords[:, 2]
    tn = (hash_tables.astype(jnp.float32)
          .reshape(N_LEVELS, HASHMAP // 128, 128, F_PER)
          .transpose(0, 1, 3, 2)
          .reshape(TBLW))
    table2 = _intl_call(tn)
    resrep = jnp.broadcast_to(
        jnp.asarray(RESOLUTIONS)[:, None], (N_LEVELS, 16)).astype(jnp.float32)
    enc4 = _enc_call(resrep, xs, ys, zs, table2)
    enc = enc4.transpose(0, 2, 1, 3).reshape(2 * N_LEVELS, N_POINTS)
    out = _mlp_call(enc,
                    W0.T.astype(jnp.float32), b0[:, None].astype(jnp.float32),
                    W1.T.astype(jnp.float32), b1[:, None].astype(jnp.float32),
                    W2.T.astype(jnp.float32), b2[:, None].astype(jnp.float32),
                    W3.T.astype(jnp.float32), b3[:, None].astype(jnp.float32))
    return out.reshape(N_POINTS, 1)


# speedup vs baseline: 11.0241x; 1.6332x over previous
import functools

import numpy as np
import jax
import jax.numpy as jnp
from jax import lax
from jax.experimental import pallas as pl
from jax.experimental.pallas import tpu as pltpu
from jax.experimental.pallas import tpu_sc as plsc

N_POINTS = 524288
N_LEVELS = 16
F_PER = 2
HASHMAP = 2 ** 19
MASK = np.int32(HASHMAP - 1)
BASE = 16
FINEST = 512
_b = np.exp((np.log(FINEST) - np.log(BASE)) / (N_LEVELS - 1))
RESOLUTIONS = np.array([int(np.ceil(BASE * _b ** i)) for i in range(N_LEVELS)],
                       dtype=np.float32)
P1 = np.int32(np.uint32(2654435761 & 0xFFFFFFFF))
P2 = np.int32(805459861)
CLIP_HI = np.float32(1.0 - 1e-06)

NC = 2
NS = 16
NW = NC * NS
PW = N_POINTS // NW
C = 128
NCHUNK = PW // C
G16 = C // 16

TBLW = N_LEVELS * HASHMAP * F_PER
BPW = TBLW // NW
IBLK = 4096
NIB = BPW // IBLK

F32 = jnp.float32
I32 = jnp.int32


def _i32(x):
    return jnp.int32(x)


def _worker_id():
    cid = lax.axis_index("c").astype(I32)
    sid = lax.axis_index("s").astype(I32)
    return sid * _i32(NC) + cid


def _intl_body(tn_hbm, tout_hbm, buf, obuf):
    wid = _worker_id()
    woff0 = wid * _i32(BPW)
    iota16 = jnp.arange(16, dtype=I32)
    zeros16 = jnp.zeros((16,), I32)
    ones16 = jnp.ones((16,), I32)

    def ib(i, carry):
        woff = woff0 + i * _i32(IBLK)
        pltpu.sync_copy(tn_hbm.at[pl.ds(woff, IBLK)], buf)

        def grp(q, carry2):
            s0 = lax.shift_right_logical(q, _i32(3)) * _i32(256) \
                + (q & _i32(7)) * _i32(16)
            f0 = buf[pl.ds(s0, 16)]
            f1 = buf[pl.ds(s0 + _i32(128), 16)]
            eidx = q * _i32(16) + iota16
            plsc.store_scatter(obuf, [eidx, zeros16], f0)
            plsc.store_scatter(obuf, [eidx, ones16], f1)
            return carry2

        lax.fori_loop(_i32(0), _i32(IBLK // 32), grp, _i32(0))
        ebase = lax.shift_right_logical(woff, _i32(1))
        pltpu.sync_copy(obuf, tout_hbm.at[pl.ds(ebase, IBLK // 2)])
        return carry

    lax.fori_loop(_i32(0), _i32(NIB), ib, _i32(0))


_intl_call = functools.partial(
    pl.kernel,
    out_type=jax.ShapeDtypeStruct((N_LEVELS * HASHMAP, F_PER), jnp.float32),
    mesh=plsc.VectorSubcoreMesh(core_axis_name="c", subcore_axis_name="s"),
    compiler_params=pltpu.CompilerParams(needs_layout_passes=False,
                                         use_tc_tiling_on_sc=False),
    scratch_types=[
        pltpu.VMEM((IBLK,), F32),
        pltpu.VMEM((IBLK // 2, F_PER), F32),
    ],
)(_intl_body)


def _enc_body(coords_hbm, table_hbm, enc_hbm,
              cv, idxb, wb, rows, encb, sem0, sem1, osem):
    wid = _worker_id()
    base0 = wid * _i32(PW)
    ct0 = wid * _i32(NCHUNK)
    sems = (sem0, sem1)

    iota16 = jnp.arange(16, dtype=I32)
    zeros16 = jnp.zeros((16,), I32)
    ones16 = jnp.ones((16,), I32)

    def chunk_body(ci, carry):
        base = base0 + ci * _i32(C)
        pltpu.sync_copy(coords_hbm.at[:, pl.ds(base, C)], cv)

        def hash_fire(l):
            b = np.int32(l & 1)
            res = RESOLUTIONS[l]
            loff = np.int32(l * HASHMAP)

            def hash_group(pg, carry3):
                sl = pl.ds(pg * _i32(16), 16)
                x = jnp.clip(cv[np.int32(0), sl], F32(0.0), CLIP_HI)
                y = jnp.clip(cv[np.int32(1), sl], F32(0.0), CLIP_HI)
                z = jnp.clip(cv[np.int32(2), sl], F32(0.0), CLIP_HI)
                sx = x * res
                sy = y * res
                sz = z * res
                ix = sx.astype(I32)
                iy = sy.astype(I32)
                iz = sz.astype(I32)
                wx1 = sx - ix.astype(F32)
                wy1 = sy - iy.astype(F32)
                wz1 = sz - iz.astype(F32)
                wx0 = F32(1.0) - wx1
                wy0 = F32(1.0) - wy1
                wz0 = F32(1.0) - wz1
                hx = (ix, ix + _i32(1))
                hy = (iy * P1, iy * P1 + P1)
                hz = (iz * P2, iz * P2 + P2)
                wx = (wx0, wx1)
                wy = (wy0, wy1)
                wz = (wz0, wz1)
                for i in range(2):
                    for j in range(2):
                        for k in range(2):
                            corner = np.int32(i * 4 + j * 2 + k)
                            hidx = ((hx[i] ^ hy[j] ^ hz[k]) & MASK) + loff
                            idxb[b, corner, sl] = hidx
                            wb[b, corner, sl] = wx[i] * wy[j] * wz[k]
                return carry3

            lax.fori_loop(_i32(0), _i32(G16), hash_group, _i32(0))
            return [pltpu.async_copy(
                        table_hbm.at[idxb.at[b, np.int32(corner)]],
                        rows.at[b, np.int32(corner)], sems[l & 1])
                    for corner in range(8)]

        def interp(l):
            b = np.int32(l & 1)

            def interp_group(pg, carry3):
                sl = pl.ds(pg * _i32(16), 16)
                p_idx = pg * _i32(16) + iota16
                bsplat = jnp.full((16,), l & 1, I32)
                acc0 = jnp.zeros((16,), F32)
                acc1 = jnp.zeros((16,), F32)
                for corner in range(8):
                    csplat = jnp.full((16,), corner, I32)
                    f0 = plsc.load_gather(rows, [bsplat, csplat, p_idx,
                                                 zeros16])
                    f1 = plsc.load_gather(rows, [bsplat, csplat, p_idx,
                                                 ones16])
                    wwv = wb[b, np.int32(corner), sl]
                    acc0 = acc0 + wwv * f0
                    acc1 = acc1 + wwv * f1
                encb[np.int32(2 * l), sl] = acc0
                encb[np.int32(2 * l + 1), sl] = acc1
                return carry3

            lax.fori_loop(_i32(0), _i32(G16), interp_group, _i32(0))

        handles = hash_fire(0)
        for l in range(N_LEVELS):
            nxt = hash_fire(l + 1) if l + 1 < N_LEVELS else None
            for cp in handles:
                cp.wait()
            interp(l)
            handles = nxt

        ct = ct0 + ci
        ocopies = [pltpu.async_copy(encb.at[pl.ds(np.int32(8 * t), 8)],
                                    enc_hbm.at[np.int32(t), ct], osem)
                   for t in range(4)]
        for cp in ocopies:
            cp.wait()
        return carry

    lax.fori_loop(_i32(0), _i32(NCHUNK), chunk_body, _i32(0))


_enc_call = functools.partial(
    pl.kernel,
    out_type=jax.ShapeDtypeStruct((4, N_POINTS // 128, 8, 128), jnp.float32),
    mesh=plsc.VectorSubcoreMesh(core_axis_name="c", subcore_axis_name="s"),
    compiler_params=pltpu.CompilerParams(needs_layout_passes=False,
                                         use_tc_tiling_on_sc=False),
    scratch_types=[
        pltpu.VMEM((3, C), F32),
        pltpu.VMEM((2, 8, C), I32),
        pltpu.VMEM((2, 8, C), F32),
        pltpu.VMEM((2, 8, C, F_PER), F32),
        pltpu.VMEM((2 * N_LEVELS, C), F32),
        pltpu.SemaphoreType.DMA,
        pltpu.SemaphoreType.DMA,
        pltpu.SemaphoreType.DMA,
    ],
)(_enc_body)


B_MLP = 2048


def _mlp_body(enc_ref, w0, b0, w1, b1, w2, b2, w3, b3, out_ref):
    h = jnp.dot(w0[...], enc_ref[...], preferred_element_type=F32) + b0[...]
    h = jnp.maximum(h, F32(0.0))
    h = jnp.dot(w1[...], h, preferred_element_type=F32) + b1[...]
    h = jnp.maximum(h, F32(0.0))
    h = jnp.dot(w2[...], h, preferred_element_type=F32) + b2[...]
    h = jnp.maximum(h, F32(0.0))
    o = jnp.dot(w3[...], h, preferred_element_type=F32) + b3[...]
    out_ref[...] = jax.nn.sigmoid(o)


IN_DIM = 2 * N_LEVELS
HIDDEN = 64

_Z = np.int32(0)


def _col_map(i):
    return (_Z, i)


def _fix_map(i):
    return (_Z, _Z)


_mlp_call = pl.pallas_call(
    _mlp_body,
    grid=(N_POINTS // B_MLP,),
    in_specs=[
        pl.BlockSpec((IN_DIM, B_MLP), _col_map),
        pl.BlockSpec((HIDDEN, IN_DIM), _fix_map),
        pl.BlockSpec((HIDDEN, 1), _fix_map),
        pl.BlockSpec((HIDDEN, HIDDEN), _fix_map),
        pl.BlockSpec((HIDDEN, 1), _fix_map),
        pl.BlockSpec((HIDDEN, HIDDEN), _fix_map),
        pl.BlockSpec((HIDDEN, 1), _fix_map),
        pl.BlockSpec((1, HIDDEN), _fix_map),
        pl.BlockSpec((1, 1), _fix_map),
    ],
    out_specs=pl.BlockSpec((1, B_MLP), _col_map),
    out_shape=jax.ShapeDtypeStruct((1, N_POINTS), jnp.float32),
)


def kernel(coords, hash_tables, W0, b0, W1, b1, W2, b2, W3, b3):
    coordsT = coords.astype(jnp.float32).T
    tn = (hash_tables.astype(jnp.float32)
          .reshape(N_LEVELS, HASHMAP // 128, 128, F_PER)
          .transpose(0, 1, 3, 2)
          .reshape(TBLW))
    table2 = _intl_call(tn)
    enc4 = _enc_call(coordsT, table2)
    enc = enc4.transpose(0, 2, 1, 3).reshape(2 * N_LEVELS, N_POINTS)
    out = _mlp_call(enc,
                    W0.T.astype(jnp.float32), b0[:, None].astype(jnp.float32),
                    W1.T.astype(jnp.float32), b1[:, None].astype(jnp.float32),
                    W2.T.astype(jnp.float32), b2[:, None].astype(jnp.float32),
                    W3.T.astype(jnp.float32), b3[:, None].astype(jnp.float32))
    return out.reshape(N_POINTS, 1)


# speedup vs baseline: 11.1248x; 1.0091x over previous
import functools

import numpy as np
import jax
import jax.numpy as jnp
from jax import lax
from jax.experimental import pallas as pl
from jax.experimental.pallas import tpu as pltpu
from jax.experimental.pallas import tpu_sc as plsc

N_POINTS = 524288
N_LEVELS = 16
F_PER = 2
HASHMAP = 2 ** 19
MASK = np.int32(HASHMAP - 1)
BASE = 16
FINEST = 512
_b = np.exp((np.log(FINEST) - np.log(BASE)) / (N_LEVELS - 1))
RESOLUTIONS = np.array([int(np.ceil(BASE * _b ** i)) for i in range(N_LEVELS)],
                       dtype=np.float32)
P1 = np.int32(np.uint32(2654435761 & 0xFFFFFFFF))
P2 = np.int32(805459861)
CLIP_HI = np.float32(1.0 - 1e-06)

NC = 2
NS = 16
NW = NC * NS
PW = N_POINTS // NW
C = 128
NCHUNK = PW // C
G16 = C // 16

TBLW = N_LEVELS * HASHMAP * F_PER
BPW = TBLW // NW
IBLK = 4096
NIB = BPW // IBLK

F32 = jnp.float32
I32 = jnp.int32


def _i32(x):
    return jnp.int32(x)


def _worker_id():
    cid = lax.axis_index("c").astype(I32)
    sid = lax.axis_index("s").astype(I32)
    return sid * _i32(NC) + cid


def _intl_body(tn_hbm, tout_hbm, buf, obuf):
    wid = _worker_id()
    woff0 = wid * _i32(BPW)
    iota16 = jnp.arange(16, dtype=I32)
    zeros16 = jnp.zeros((16,), I32)
    ones16 = jnp.ones((16,), I32)

    def ib(i, carry):
        woff = woff0 + i * _i32(IBLK)
        pltpu.sync_copy(tn_hbm.at[pl.ds(woff, IBLK)], buf)

        def grp(q, carry2):
            s0 = lax.shift_right_logical(q, _i32(3)) * _i32(256) \
                + (q & _i32(7)) * _i32(16)
            f0 = buf[pl.ds(s0, 16)]
            f1 = buf[pl.ds(s0 + _i32(128), 16)]
            eidx = q * _i32(16) + iota16
            plsc.store_scatter(obuf, [eidx, zeros16], f0)
            plsc.store_scatter(obuf, [eidx, ones16], f1)
            return carry2

        lax.fori_loop(_i32(0), _i32(IBLK // 32), grp, _i32(0))
        ebase = lax.shift_right_logical(woff, _i32(1))
        pltpu.sync_copy(obuf, tout_hbm.at[pl.ds(ebase, IBLK // 2)])
        return carry

    lax.fori_loop(_i32(0), _i32(NIB), ib, _i32(0))


_intl_call = functools.partial(
    pl.kernel,
    out_type=jax.ShapeDtypeStruct((N_LEVELS * HASHMAP, F_PER), jnp.float32),
    mesh=plsc.VectorSubcoreMesh(core_axis_name="c", subcore_axis_name="s"),
    compiler_params=pltpu.CompilerParams(needs_layout_passes=False,
                                         use_tc_tiling_on_sc=False),
    scratch_types=[
        pltpu.VMEM((IBLK,), F32),
        pltpu.VMEM((IBLK // 2, F_PER), F32),
    ],
)(_intl_body)


def _enc_body(coords_hbm, table_hbm, enc_hbm,
              cv, idxb, wb, rows, encb, sem0, sem1, osem):
    wid = _worker_id()
    base0 = wid * _i32(PW)
    ct0 = wid * _i32(NCHUNK)
    sems = (sem0, sem1)

    iota16 = jnp.arange(16, dtype=I32)
    zeros16 = jnp.zeros((16,), I32)
    ones16 = jnp.ones((16,), I32)

    def chunk_body(ci, carry):
        base = base0 + ci * _i32(C)
        pltpu.sync_copy(coords_hbm.at[:, pl.ds(base, C)], cv)

        def clip_group(pg, carry3):
            sl = pl.ds(pg * _i32(16), 16)
            for r in range(3):
                cv[np.int32(r), sl] = jnp.clip(cv[np.int32(r), sl],
                                               F32(0.0), CLIP_HI)
            return carry3

        lax.fori_loop(_i32(0), _i32(G16), clip_group, _i32(0))

        def hash_fire(l):
            b = np.int32(l & 1)
            res = RESOLUTIONS[l]
            loff = np.int32(l * HASHMAP)

            def hash_group(pg, carry3):
                sl = pl.ds(pg * _i32(16), 16)
                sx = cv[np.int32(0), sl] * res
                sy = cv[np.int32(1), sl] * res
                sz = cv[np.int32(2), sl] * res
                ix = sx.astype(I32)
                iy = sy.astype(I32)
                iz = sz.astype(I32)
                wb[b, np.int32(0), sl] = sx - ix.astype(F32)
                wb[b, np.int32(1), sl] = sy - iy.astype(F32)
                wb[b, np.int32(2), sl] = sz - iz.astype(F32)
                hy0 = iy * P1
                hz0 = iz * P2
                hx = (ix, ix + _i32(1))
                hy = (hy0, hy0 + P1)
                hz = (hz0, hz0 + P2)
                for i in range(2):
                    for j in range(2):
                        hxy = hx[i] ^ hy[j]
                        for k in range(2):
                            corner = np.int32(i * 4 + j * 2 + k)
                            idxb[b, corner, sl] = ((hxy ^ hz[k]) & MASK) + loff
                return carry3

            lax.fori_loop(_i32(0), _i32(G16), hash_group, _i32(0))
            return [pltpu.async_copy(
                        table_hbm.at[idxb.at[b, np.int32(corner)]],
                        rows.at[pl.ds(np.int32(((l & 1) * 8 + corner) * C),
                                      C)],
                        sems[l & 1])
                    for corner in range(8)]

        def interp(l):
            b = np.int32(l & 1)
            rbase = np.int32((l & 1) * 8 * C)

            def interp_group(pg, carry3):
                sl = pl.ds(pg * _i32(16), 16)
                p_idx = rbase + pg * _i32(16) + iota16
                wx1 = wb[b, np.int32(0), sl]
                wy1 = wb[b, np.int32(1), sl]
                wz1 = wb[b, np.int32(2), sl]
                wx = (F32(1.0) - wx1, wx1)
                wy = (F32(1.0) - wy1, wy1)
                wz = (F32(1.0) - wz1, wz1)
                acc0 = jnp.zeros((16,), F32)
                acc1 = jnp.zeros((16,), F32)
                for i in range(2):
                    for j in range(2):
                        wxy = wx[i] * wy[j]
                        for k in range(2):
                            corner = i * 4 + j * 2 + k
                            r_idx = p_idx + np.int32(corner * C)
                            f0 = plsc.load_gather(rows, [r_idx, zeros16])
                            f1 = plsc.load_gather(rows, [r_idx, ones16])
                            ww = wxy * wz[k]
                            acc0 = acc0 + ww * f0
                            acc1 = acc1 + ww * f1
                encb[np.int32(2 * l), sl] = acc0
                encb[np.int32(2 * l + 1), sl] = acc1
                return carry3

            lax.fori_loop(_i32(0), _i32(G16), interp_group, _i32(0))

        handles = hash_fire(0)
        for l in range(N_LEVELS):
            nxt = hash_fire(l + 1) if l + 1 < N_LEVELS else None
            for cp in handles:
                cp.wait()
            interp(l)
            handles = nxt

        ct = ct0 + ci
        ocopies = [pltpu.async_copy(encb.at[pl.ds(np.int32(8 * t), 8)],
                                    enc_hbm.at[np.int32(t), ct], osem)
                   for t in range(4)]
        for cp in ocopies:
            cp.wait()
        return carry

    lax.fori_loop(_i32(0), _i32(NCHUNK), chunk_body, _i32(0))


_enc_call = functools.partial(
    pl.kernel,
    out_type=jax.ShapeDtypeStruct((4, N_POINTS // 128, 8, 128), jnp.float32),
    mesh=plsc.VectorSubcoreMesh(core_axis_name="c", subcore_axis_name="s"),
    compiler_params=pltpu.CompilerParams(needs_layout_passes=False,
                                         use_tc_tiling_on_sc=False),
    scratch_types=[
        pltpu.VMEM((3, C), F32),
        pltpu.VMEM((2, 8, C), I32),
        pltpu.VMEM((2, 3, C), F32),
        pltpu.VMEM((2 * 8 * C, F_PER), F32),
        pltpu.VMEM((2 * N_LEVELS, C), F32),
        pltpu.SemaphoreType.DMA,
        pltpu.SemaphoreType.DMA,
        pltpu.SemaphoreType.DMA,
    ],
)(_enc_body)


B_MLP = 2048


def _mlp_body(enc_ref, w0, b0, w1, b1, w2, b2, w3, b3, out_ref):
    h = jnp.dot(w0[...], enc_ref[...], preferred_element_type=F32) + b0[...]
    h = jnp.maximum(h, F32(0.0))
    h = jnp.dot(w1[...], h, preferred_element_type=F32) + b1[...]
    h = jnp.maximum(h, F32(0.0))
    h = jnp.dot(w2[...], h, preferred_element_type=F32) + b2[...]
    h = jnp.maximum(h, F32(0.0))
    o = jnp.dot(w3[...], h, preferred_element_type=F32) + b3[...]
    out_ref[...] = jax.nn.sigmoid(o)


IN_DIM = 2 * N_LEVELS
HIDDEN = 64

_Z = np.int32(0)


def _col_map(i):
    return (_Z, i)


def _fix_map(i):
    return (_Z, _Z)


_mlp_call = pl.pallas_call(
    _mlp_body,
    grid=(N_POINTS // B_MLP,),
    in_specs=[
        pl.BlockSpec((IN_DIM, B_MLP), _col_map),
        pl.BlockSpec((HIDDEN, IN_DIM), _fix_map),
        pl.BlockSpec((HIDDEN, 1), _fix_map),
        pl.BlockSpec((HIDDEN, HIDDEN), _fix_map),
        pl.BlockSpec((HIDDEN, 1), _fix_map),
        pl.BlockSpec((HIDDEN, HIDDEN), _fix_map),
        pl.BlockSpec((HIDDEN, 1), _fix_map),
        pl.BlockSpec((1, HIDDEN), _fix_map),
        pl.BlockSpec((1, 1), _fix_map),
    ],
    out_specs=pl.BlockSpec((1, B_MLP), _col_map),
    out_shape=jax.ShapeDtypeStruct((1, N_POINTS), jnp.float32),
)


def kernel(coords, hash_tables, W0, b0, W1, b1, W2, b2, W3, b3):
    coordsT = coords.astype(jnp.float32).T
    tn = (hash_tables.astype(jnp.float32)
          .reshape(N_LEVELS, HASHMAP // 128, 128, F_PER)
          .transpose(0, 1, 3, 2)
          .reshape(TBLW))
    table2 = _intl_call(tn)
    enc4 = _enc_call(coordsT, table2)
    enc = enc4.transpose(0, 2, 1, 3).reshape(2 * N_LEVELS, N_POINTS)
    out = _mlp_call(enc,
                    W0.T.astype(jnp.float32), b0[:, None].astype(jnp.float32),
                    W1.T.astype(jnp.float32), b1[:, None].astype(jnp.float32),
                    W2.T.astype(jnp.float32), b2[:, None].astype(jnp.float32),
                    W3.T.astype(jnp.float32), b3[:, None].astype(jnp.float32))
    return out.reshape(N_POINTS, 1)


# speedup vs baseline: 11.1370x; 1.0011x over previous
import functools

import numpy as np
import jax
import jax.numpy as jnp
from jax import lax
from jax.experimental import pallas as pl
from jax.experimental.pallas import tpu as pltpu
from jax.experimental.pallas import tpu_sc as plsc

N_POINTS = 524288
N_LEVELS = 16
F_PER = 2
HASHMAP = 2 ** 19
MASK = np.int32(HASHMAP - 1)
BASE = 16
FINEST = 512
_b = np.exp((np.log(FINEST) - np.log(BASE)) / (N_LEVELS - 1))
RESOLUTIONS = np.array([int(np.ceil(BASE * _b ** i)) for i in range(N_LEVELS)],
                       dtype=np.float32)
P1 = np.int32(np.uint32(2654435761 & 0xFFFFFFFF))
P2 = np.int32(805459861)
CLIP_HI = np.float32(1.0 - 1e-06)

NC = 2
NS = 16
NW = NC * NS
PW = N_POINTS // NW
C = 128
NCHUNK = PW // C
G16 = C // 16

TBLW = N_LEVELS * HASHMAP * F_PER
BPW = TBLW // NW
IBLK = 4096
NIB = BPW // IBLK

F32 = jnp.float32
I32 = jnp.int32


def _i32(x):
    return jnp.int32(x)


def _worker_id():
    cid = lax.axis_index("c").astype(I32)
    sid = lax.axis_index("s").astype(I32)
    return sid * _i32(NC) + cid


def _intl_body(tn_hbm, tout_hbm, buf, obuf):
    wid = _worker_id()
    woff0 = wid * _i32(BPW)
    iota16 = jnp.arange(16, dtype=I32)
    zeros16 = jnp.zeros((16,), I32)
    ones16 = jnp.ones((16,), I32)

    def ib(i, carry):
        woff = woff0 + i * _i32(IBLK)
        pltpu.sync_copy(tn_hbm.at[pl.ds(woff, IBLK)], buf)

        def grp(q, carry2):
            s0 = lax.shift_right_logical(q, _i32(3)) * _i32(256) \
                + (q & _i32(7)) * _i32(16)
            f0 = buf[pl.ds(s0, 16)]
            f1 = buf[pl.ds(s0 + _i32(128), 16)]
            eidx = q * _i32(16) + iota16
            plsc.store_scatter(obuf, [eidx, zeros16], f0)
            plsc.store_scatter(obuf, [eidx, ones16], f1)
            return carry2

        lax.fori_loop(_i32(0), _i32(IBLK // 32), grp, _i32(0))
        ebase = lax.shift_right_logical(woff, _i32(1))
        pltpu.sync_copy(obuf, tout_hbm.at[pl.ds(ebase, IBLK // 2)])
        return carry

    lax.fori_loop(_i32(0), _i32(NIB), ib, _i32(0))


_intl_call = functools.partial(
    pl.kernel,
    out_type=jax.ShapeDtypeStruct((N_LEVELS * HASHMAP, F_PER), jnp.float32),
    mesh=plsc.VectorSubcoreMesh(core_axis_name="c", subcore_axis_name="s"),
    compiler_params=pltpu.CompilerParams(needs_layout_passes=False,
                                         use_tc_tiling_on_sc=False),
    scratch_types=[
        pltpu.VMEM((IBLK,), F32),
        pltpu.VMEM((IBLK // 2, F_PER), F32),
    ],
)(_intl_body)


def _enc_body(coords_hbm, table_hbm, enc_hbm,
              cv, idxb, wb, rows, encb, sem0, sem1, osem):
    wid = _worker_id()
    base0 = wid * _i32(PW)
    ct0 = wid * _i32(NCHUNK)
    sems = (sem0, sem1)

    iota16 = jnp.arange(16, dtype=I32)
    zeros16 = jnp.zeros((16,), I32)
    ones16 = jnp.ones((16,), I32)

    def chunk_body(ci, carry):
        base = base0 + ci * _i32(C)
        pltpu.sync_copy(coords_hbm.at[:, pl.ds(base, C)], cv)

        def clip_group(pg, carry3):
            sl = pl.ds(pg * _i32(16), 16)
            for r in range(3):
                cv[np.int32(r), sl] = jnp.clip(cv[np.int32(r), sl],
                                               F32(0.0), CLIP_HI)
            return carry3

        lax.fori_loop(_i32(0), _i32(G16), clip_group, _i32(0))

        def hash_fire(l):
            b = np.int32(l & 1)
            res = RESOLUTIONS[l]
            loff = np.int32(l * HASHMAP)

            def hash_group(pg, carry3):
                sl = pl.ds(pg * _i32(16), 16)
                sx = cv[np.int32(0), sl] * res
                sy = cv[np.int32(1), sl] * res
                sz = cv[np.int32(2), sl] * res
                ix = sx.astype(I32)
                iy = sy.astype(I32)
                iz = sz.astype(I32)
                wb[b, np.int32(0), sl] = sx - ix.astype(F32)
                wb[b, np.int32(1), sl] = sy - iy.astype(F32)
                wb[b, np.int32(2), sl] = sz - iz.astype(F32)
                hy0 = iy * P1
                hz0 = iz * P2
                hx = (ix, ix + _i32(1))
                hy = (hy0, hy0 + P1)
                hz = (hz0, hz0 + P2)
                for i in range(2):
                    for j in range(2):
                        hxy = hx[i] ^ hy[j]
                        for k in range(2):
                            corner = np.int32(i * 4 + j * 2 + k)
                            idxb[b, corner, sl] = ((hxy ^ hz[k]) & MASK) + loff
                return carry3

            lax.fori_loop(_i32(0), _i32(G16), hash_group, _i32(0))
            return [pltpu.async_copy(
                        table_hbm.at[idxb.at[b, np.int32(corner)]],
                        rows.at[b, np.int32(corner)], sems[l & 1])
                    for corner in range(8)]

        def interp(l):
            b = np.int32(l & 1)
            bsplat = jnp.full((16,), l & 1, I32)

            def interp_group(pg, carry3):
                sl = pl.ds(pg * _i32(16), 16)
                p_idx = pg * _i32(16) + iota16
                wx1 = wb[b, np.int32(0), sl]
                wy1 = wb[b, np.int32(1), sl]
                wz1 = wb[b, np.int32(2), sl]
                wx = (F32(1.0) - wx1, wx1)
                wy = (F32(1.0) - wy1, wy1)
                wz = (F32(1.0) - wz1, wz1)
                acc0 = jnp.zeros((16,), F32)
                acc1 = jnp.zeros((16,), F32)
                for i in range(2):
                    for j in range(2):
                        wxy = wx[i] * wy[j]
                        for k in range(2):
                            corner = i * 4 + j * 2 + k
                            csplat = jnp.full((16,), corner, I32)
                            f0 = plsc.load_gather(rows, [bsplat, csplat,
                                                         p_idx, zeros16])
                            f1 = plsc.load_gather(rows, [bsplat, csplat,
                                                         p_idx, ones16])
                            ww = wxy * wz[k]
                            acc0 = acc0 + ww * f0
                            acc1 = acc1 + ww * f1
                encb[np.int32(2 * l), sl] = acc0
                encb[np.int32(2 * l + 1), sl] = acc1
                return carry3

            lax.fori_loop(_i32(0), _i32(G16), interp_group, _i32(0))

        handles = hash_fire(0)
        for l in range(N_LEVELS):
            nxt = hash_fire(l + 1) if l + 1 < N_LEVELS else None
            for cp in handles:
                cp.wait()
            interp(l)
            handles = nxt

        ct = ct0 + ci
        ocopies = [pltpu.async_copy(encb.at[pl.ds(np.int32(8 * t), 8)],
                                    enc_hbm.at[np.int32(t), ct], osem)
                   for t in range(4)]
        for cp in ocopies:
            cp.wait()
        return carry

    lax.fori_loop(_i32(0), _i32(NCHUNK), chunk_body, _i32(0))


_enc_call = functools.partial(
    pl.kernel,
    out_type=jax.ShapeDtypeStruct((4, N_POINTS // 128, 8, 128), jnp.float32),
    mesh=plsc.VectorSubcoreMesh(core_axis_name="c", subcore_axis_name="s"),
    compiler_params=pltpu.CompilerParams(needs_layout_passes=False,
                                         use_tc_tiling_on_sc=False),
    scratch_types=[
        pltpu.VMEM((3, C), F32),
        pltpu.VMEM((2, 8, C), I32),
        pltpu.VMEM((2, 3, C), F32),
        pltpu.VMEM((2, 8, C, F_PER), F32),
        pltpu.VMEM((2 * N_LEVELS, C), F32),
        pltpu.SemaphoreType.DMA,
        pltpu.SemaphoreType.DMA,
        pltpu.SemaphoreType.DMA,
    ],
)(_enc_body)


B_MLP = 2048


def _mlp_body(enc_ref, w0, b0, w1, b1, w2, b2, w3, b3, out_ref):
    h = jnp.dot(w0[...], enc_ref[...], preferred_element_type=F32) + b0[...]
    h = jnp.maximum(h, F32(0.0))
    h = jnp.dot(w1[...], h, preferred_element_type=F32) + b1[...]
    h = jnp.maximum(h, F32(0.0))
    h = jnp.dot(w2[...], h, preferred_element_type=F32) + b2[...]
    h = jnp.maximum(h, F32(0.0))
    o = jnp.dot(w3[...], h, preferred_element_type=F32) + b3[...]
    out_ref[...] = jax.nn.sigmoid(o)


IN_DIM = 2 * N_LEVELS
HIDDEN = 64

_Z = np.int32(0)


def _col_map(i):
    return (_Z, i)


def _fix_map(i):
    return (_Z, _Z)


_mlp_call = pl.pallas_call(
    _mlp_body,
    grid=(N_POINTS // B_MLP,),
    in_specs=[
        pl.BlockSpec((IN_DIM, B_MLP), _col_map),
        pl.BlockSpec((HIDDEN, IN_DIM), _fix_map),
        pl.BlockSpec((HIDDEN, 1), _fix_map),
        pl.BlockSpec((HIDDEN, HIDDEN), _fix_map),
        pl.BlockSpec((HIDDEN, 1), _fix_map),
        pl.BlockSpec((HIDDEN, HIDDEN), _fix_map),
        pl.BlockSpec((HIDDEN, 1), _fix_map),
        pl.BlockSpec((1, HIDDEN), _fix_map),
        pl.BlockSpec((1, 1), _fix_map),
    ],
    out_specs=pl.BlockSpec((1, B_MLP), _col_map),
    out_shape=jax.ShapeDtypeStruct((1, N_POINTS), jnp.float32),
)


def kernel(coords, hash_tables, W0, b0, W1, b1, W2, b2, W3, b3):
    coordsT = coords.astype(jnp.float32).T
    tn = (hash_tables.astype(jnp.float32)
          .reshape(N_LEVELS, HASHMAP // 128, 128, F_PER)
          .transpose(0, 1, 3, 2)
          .reshape(TBLW))
    table2 = _intl_call(tn)
    enc4 = _enc_call(coordsT, table2)
    enc = enc4.transpose(0, 2, 1, 3).reshape(2 * N_LEVELS, N_POINTS)
    out = _mlp_call(enc,
                    W0.T.astype(jnp.float32), b0[:, None].astype(jnp.float32),
                    W1.T.astype(jnp.float32), b1[:, None].astype(jnp.float32),
                    W2.T.astype(jnp.float32), b2[:, None].astype(jnp.float32),
                    W3.T.astype(jnp.float32), b3[:, None].astype(jnp.float32))
    return out.reshape(N_POINTS, 1)
